# baseline (speedup 1.0000x reference)
import functools
import jax
import jax.numpy as jnp
from jax.experimental import pallas as pl
from jax.experimental.pallas import tpu as pltpu

N = 10000
DEG = 32
D = 128
BN = 200


def _tc_body(alpha_ref, msg_ref, ce_ref, out_ref):
    acc = ce_ref[:, :]
    for k in range(DEG):
        acc = acc + alpha_ref[:, k][:, None] * msg_ref[:, k, :]
    out_ref[:, :] = acc


def kernel(curr_emb, alpha, msg):
    alpha2 = alpha[:, :, 0]
    ce = curr_emb[:, 0, :]
    grid = (N // BN,)
    return pl.pallas_call(
        _tc_body,
        grid=grid,
        in_specs=[
            pl.BlockSpec((BN, DEG), lambda i: (i, 0)),
            pl.BlockSpec((BN, DEG, D), lambda i: (i, 0, 0)),
            pl.BlockSpec((BN, D), lambda i: (i, 0)),
        ],
        out_specs=pl.BlockSpec((BN, D), lambda i: (i, 0)),
        out_shape=jax.ShapeDtypeStruct((N, D), jnp.float32),
    )(alpha2, msg, ce)
